# Initial kernel scaffold; baseline (speedup 1.0000x reference)
import jax
import jax.numpy as jnp
from jax.experimental import pallas as pl


def kernel(x, edge_index):
    raise NotImplementedError("write your pallas kernel here")


# speedup vs baseline: 2.4490x; 2.4490x over previous
import jax
import jax.numpy as jnp
from jax import lax
from jax.experimental import pallas as pl
from jax.experimental.pallas import tpu as pltpu
from jax.experimental.pallas import tpu_sc as plsc

N = 10000
D = 128
E = 320000
NS = 16
NP = 10240
RP = NP // NS
RB = RP // 64
ET = E // NS
EP = 20480
NCH = EP // 128
NG = NCH // 8
STEPS = 5
L = 16


def _rsqrt16(v):
    s = 0.5 * (v + 1.0)
    for _ in range(14):
        s = 0.5 * (s + v / s)
    return jnp.where(v == 0.0, jnp.float32(jnp.inf), 1.0 / s)


def _body(xp, srcs, dsts, y_hbm, h_hbm, dinv_hbm,
          acc, src_v, dstb, msgs, cbuf, zbuf, gsem):
    wid = lax.axis_index("s")
    row0 = wid * RP

    pltpu.sync_copy(srcs.at[wid], src_v)

    zeros = jnp.zeros((L,), jnp.float32)
    ones = jnp.ones((L,), jnp.float32)

    @pl.loop(0, 16)
    def _fz(r):
        for s in range(8):
            zbuf[r, pl.ds(s * L, L)] = zeros

    @pl.loop(0, 128)
    def _fo(r):
        for s in range(8):
            msgs[r, pl.ds(s * L, L)] = ones

    @pl.loop(0, RP // 16)
    def _z0(i):
        pltpu.sync_copy(zbuf, acc.at[pl.ds(row0 + i * 16, 16)])
    plsc.subcore_barrier()

    @pl.loop(0, NG)
    def _deg(g):
        pltpu.sync_copy(dsts.at[wid, pl.ds(g * 8, 8)], dstb)
        for j in range(8):
            pltpu.sync_copy(msgs, acc.at[dstb.at[j]], add=True)
    plsc.subcore_barrier()

    @pl.loop(0, RB)
    def _init(ch):
        r0 = row0 + ch * 64
        pltpu.sync_copy(acc.at[pl.ds(r0, 64)], msgs.at[pl.ds(0, 64)])
        pltpu.sync_copy(xp.at[pl.ds(r0, 64)], cbuf)

        @pl.loop(0, 64)
        def _cv(r):
            dv = _rsqrt16(msgs[r, pl.ds(0, L)])
            for s in range(8):
                sl = pl.ds(s * L, L)
                msgs[64 + r, sl] = dv
                cbuf[r, sl] = cbuf[r, sl] * dv

        pltpu.sync_copy(msgs.at[pl.ds(64, 64)], dinv_hbm.at[pl.ds(r0, 64)])
        pltpu.sync_copy(cbuf, h_hbm.at[pl.ds(r0, 64)])
        for z in range(4):
            pltpu.sync_copy(zbuf, acc.at[pl.ds(r0 + z * 16, 16)])
    plsc.subcore_barrier()

    @pl.loop(0, STEPS)
    def _step(step):
        @pl.loop(0, NG)
        def _edge(g):
            pltpu.sync_copy(dsts.at[wid, pl.ds(g * 8, 8)], dstb)
            for j in range(8):
                idx = src_v.at[pl.ds((g * 8 + j) * 128, 128)]
                pltpu.async_copy(h_hbm.at[idx], msgs, gsem).wait()
                pltpu.sync_copy(msgs, acc.at[dstb.at[j]], add=True)
        plsc.subcore_barrier()

        @pl.loop(0, RB)
        def _comb(ch):
            r0 = row0 + ch * 64
            pltpu.sync_copy(acc.at[pl.ds(r0, 64)], msgs.at[pl.ds(0, 64)])
            pltpu.sync_copy(dinv_hbm.at[pl.ds(r0, 64)], msgs.at[pl.ds(64, 64)])
            pltpu.sync_copy(xp.at[pl.ds(r0, 64)], cbuf)

            @pl.loop(0, 64)
            def _cv(r):
                for s in range(8):
                    sl = pl.ds(s * L, L)
                    yv = (0.5 * (msgs[r, sl] * msgs[64 + r, sl])
                          + 0.5 * cbuf[r, sl])
                    cbuf[r, sl] = yv
                    msgs[r, sl] = yv * msgs[64 + r, sl]

            pltpu.sync_copy(cbuf, y_hbm.at[pl.ds(r0, 64)])
            pltpu.sync_copy(msgs.at[pl.ds(0, 64)], h_hbm.at[pl.ds(r0, 64)])
            for z in range(4):
                pltpu.sync_copy(zbuf, acc.at[pl.ds(r0 + z * 16, 16)])
        plsc.subcore_barrier()


def _make_kernel():
    f32 = jnp.float32
    return pl.kernel(
        _body,
        out_type=(
            jax.ShapeDtypeStruct((NP, D), f32),
            jax.ShapeDtypeStruct((NP, D), f32),
            jax.ShapeDtypeStruct((NP, D), f32),
        ),
        mesh=plsc.VectorSubcoreMesh(
            core_axis_name="c", subcore_axis_name="s", num_cores=1),
        scratch_types=[
            pltpu.VMEM_SHARED((NP, D), f32),
            pltpu.VMEM((EP,), jnp.int32),
            pltpu.VMEM((8, 128), jnp.int32),
            pltpu.VMEM((128, D), f32),
            pltpu.VMEM((64, D), f32),
            pltpu.VMEM((16, D), f32),
            pltpu.SemaphoreType.DMA,
        ],
    )


_kernel_call = _make_kernel()


def kernel(x, edge_index):
    src = edge_index[0].reshape(NS, ET)
    dst = edge_index[1].reshape(NS, ET)
    pad = EP - ET
    src = jnp.pad(src, ((0, 0), (0, pad)))
    dst = jnp.pad(dst, ((0, 0), (0, pad)), constant_values=N)
    dst = dst.reshape(NS, NCH, 128)
    xp = jnp.pad(x, ((0, NP - N), (0, 0)))
    y, _h, _dinv = _kernel_call(xp, src, dst)
    return y[:N]


# speedup vs baseline: 3.0322x; 1.2381x over previous
import jax
import jax.numpy as jnp
from jax import lax
from jax.experimental import pallas as pl
from jax.experimental.pallas import tpu as pltpu
from jax.experimental.pallas import tpu_sc as plsc

N = 10000
D = 128
E = 320000
NS = 16
NP = 10240
RP = NP // NS
RB = RP // 64
ET = E // NS
EP = 20480
NCH = EP // 128
GC = 16
NG = NCH // GC
STEPS = 5
L = 16


def _rsqrt16(v):
    s = 0.5 * (v + 1.0)
    for _ in range(14):
        s = 0.5 * (s + v / s)
    return jnp.where(v == 0.0, jnp.float32(jnp.inf), 1.0 / s)


def _body(xp, srcs, dsts, y_hbm, h_hbm, dinv_hbm,
          acc, srcb, dstb, msga, msgb, cbuf, zbuf,
          gsem0, gsem1, ssem0, ssem1):
    wid = lax.axis_index("s")
    row0 = wid * RP

    zeros = jnp.zeros((L,), jnp.float32)
    ones = jnp.ones((L,), jnp.float32)

    @pl.loop(0, 16)
    def _fz(r):
        for s in range(8):
            zbuf[r, pl.ds(s * L, L)] = zeros

    @pl.loop(0, 128)
    def _fo(r):
        for s in range(8):
            msga[r, pl.ds(s * L, L)] = ones

    @pl.loop(0, RP // 16)
    def _z0(i):
        pltpu.sync_copy(zbuf, acc.at[pl.ds(row0 + i * 16, 16)])
    plsc.subcore_barrier()

    @pl.loop(0, NG)
    def _deg(g):
        pltpu.sync_copy(dsts.at[wid, pl.ds(g * GC, GC)], dstb)
        descs = []
        for j in range(GC):
            descs.append(
                pltpu.async_copy(msga, acc.at[dstb.at[j]], ssem0, add=True))
        for d in descs:
            d.wait()
    plsc.subcore_barrier()

    def _combine(r0, first):
        d1 = pltpu.async_copy(acc.at[pl.ds(r0, 64)],
                              msga.at[pl.ds(0, 64)], gsem0)
        d3 = pltpu.async_copy(xp.at[pl.ds(r0, 64)], cbuf, ssem0)
        if not first:
            d2 = pltpu.async_copy(dinv_hbm.at[pl.ds(r0, 64)],
                                  msga.at[pl.ds(64, 64)], gsem1)
            d2.wait()
        d1.wait()
        d3.wait()

        if first:
            @pl.loop(0, 64)
            def _cv(r):
                dv = _rsqrt16(msga[r, pl.ds(0, L)])
                for s in range(8):
                    sl = pl.ds(s * L, L)
                    msga[64 + r, sl] = dv
                    cbuf[r, sl] = cbuf[r, sl] * dv
            pltpu.sync_copy(msga.at[pl.ds(64, 64)],
                            dinv_hbm.at[pl.ds(r0, 64)])
            pltpu.sync_copy(cbuf, h_hbm.at[pl.ds(r0, 64)])
        else:
            @pl.loop(0, 64)
            def _cv(r):
                for s in range(8):
                    sl = pl.ds(s * L, L)
                    yv = (0.5 * (msga[r, sl] * msga[64 + r, sl])
                          + 0.5 * cbuf[r, sl])
                    cbuf[r, sl] = yv
                    msga[r, sl] = yv * msga[64 + r, sl]
            pltpu.sync_copy(cbuf, y_hbm.at[pl.ds(r0, 64)])
            pltpu.sync_copy(msga.at[pl.ds(0, 64)], h_hbm.at[pl.ds(r0, 64)])

        d5 = pltpu.async_copy(zbuf, acc.at[pl.ds(r0, 16)], gsem0)
        d6 = pltpu.async_copy(zbuf, acc.at[pl.ds(r0 + 16, 16)], gsem1)
        d7 = pltpu.async_copy(zbuf, acc.at[pl.ds(r0 + 32, 16)], ssem0)
        d8 = pltpu.async_copy(zbuf, acc.at[pl.ds(r0 + 48, 16)], ssem1)
        d5.wait()
        d6.wait()
        d7.wait()
        d8.wait()

    @pl.loop(0, RB)
    def _init(ch):
        _combine(row0 + ch * 64, first=True)
    plsc.subcore_barrier()

    @pl.loop(0, STEPS)
    def _step(step):
        @pl.loop(0, NG)
        def _edge(g):
            pltpu.sync_copy(srcs.at[wid, pl.ds(g * (GC * 128), GC * 128)],
                            srcb)
            pltpu.sync_copy(dsts.at[wid, pl.ds(g * GC, GC)], dstb)
            bufs = (msga, msgb)
            gsems = (gsem0, gsem1)
            ssems = (ssem0, ssem1)
            gd = [None, None]
            sd = [None, None]
            for j in range(GC):
                b = j & 1
                if sd[b] is not None:
                    sd[b].wait()
                idx = srcb.at[pl.ds(j * 128, 128)]
                gd[b] = pltpu.async_copy(h_hbm.at[idx], bufs[b], gsems[b])
                if j >= 1:
                    pb = (j - 1) & 1
                    gd[pb].wait()
                    sd[pb] = pltpu.async_copy(
                        bufs[pb], acc.at[dstb.at[j - 1]], ssems[pb],
                        add=True)
            lb = (GC - 1) & 1
            gd[lb].wait()
            sd[lb] = pltpu.async_copy(
                bufs[lb], acc.at[dstb.at[GC - 1]], ssems[lb], add=True)
            sd[0].wait()
            sd[1].wait()
        plsc.subcore_barrier()

        @pl.loop(0, RB)
        def _comb(ch):
            _combine(row0 + ch * 64, first=False)
        plsc.subcore_barrier()


def _make_kernel():
    f32 = jnp.float32
    return pl.kernel(
        _body,
        out_type=(
            jax.ShapeDtypeStruct((NP, D), f32),
            jax.ShapeDtypeStruct((NP, D), f32),
            jax.ShapeDtypeStruct((NP, D), f32),
        ),
        mesh=plsc.VectorSubcoreMesh(
            core_axis_name="c", subcore_axis_name="s", num_cores=1),
        scratch_types=[
            pltpu.VMEM_SHARED((NP, D), f32),
            pltpu.VMEM((GC * 128,), jnp.int32),
            pltpu.VMEM((GC, 128), jnp.int32),
            pltpu.VMEM((128, D), f32),
            pltpu.VMEM((128, D), f32),
            pltpu.VMEM((64, D), f32),
            pltpu.VMEM((16, D), f32),
            pltpu.SemaphoreType.DMA,
            pltpu.SemaphoreType.DMA,
            pltpu.SemaphoreType.DMA,
            pltpu.SemaphoreType.DMA,
        ],
    )


_kernel_call = _make_kernel()


def kernel(x, edge_index):
    src = edge_index[0].reshape(NS, ET)
    dst = edge_index[1].reshape(NS, ET)
    pad = EP - ET
    src = jnp.pad(src, ((0, 0), (0, pad)))
    dst = jnp.pad(dst, ((0, 0), (0, pad)), constant_values=N)
    dst = dst.reshape(NS, NCH, 128)
    xp = jnp.pad(x, ((0, NP - N), (0, 0)))
    y, _h, _dinv = _kernel_call(xp, src, dst)
    return y[:N]


# speedup vs baseline: 3.6337x; 1.1984x over previous
import jax
import jax.numpy as jnp
from jax import lax
from jax.experimental import pallas as pl
from jax.experimental.pallas import tpu as pltpu
from jax.experimental.pallas import tpu_sc as plsc

N = 10000
D = 128
E = 320000
NC = 2
NS = 16
NW = NC * NS
NP = 10240
RPT = NP // NS
RPW = NP // NW
ET = E // NW
EP = 10240
NCH = EP // 128
GC = 16
NG = NCH // GC
STEPS = 5
L = 16

_MESH = plsc.VectorSubcoreMesh(core_axis_name="c", subcore_axis_name="s")
f32 = jnp.float32


def _rsqrt16(v):
    s = 0.5 * (v + 1.0)
    for _ in range(14):
        s = 0.5 * (s + v / s)
    return jnp.where(v == 0.0, f32(jnp.inf), 1.0 / s)


def _fill(buf, rows, val):
    v = jnp.full((L,), val, f32)

    @pl.loop(0, rows)
    def _f(r):
        for s in range(8):
            buf[r, pl.ds(s * L, L)] = v


def _zero_acc_stripe(acc, zbuf, row0):
    @pl.loop(0, RPT // 16)
    def _z0(i):
        pltpu.sync_copy(zbuf, acc.at[pl.ds(row0 + i * 16, 16)])


def _dump_acc_stripe(acc, p, cid, row0):
    @pl.loop(0, RPT // 128)
    def _dmp(i):
        r0 = row0 + i * 128
        pltpu.sync_copy(acc.at[pl.ds(r0, 128)], p.at[cid, pl.ds(r0, 128)])


def _deg_body(dsts, p, acc, dstb, ones, zbuf, sem0):
    cid = lax.axis_index("c")
    wid = lax.axis_index("s")
    gid = cid * NS + wid
    row0 = wid * RPT

    _fill(zbuf, 16, 0.0)
    _fill(ones, 128, 1.0)
    _zero_acc_stripe(acc, zbuf, row0)
    plsc.subcore_barrier()

    @pl.loop(0, NG)
    def _deg(g):
        pltpu.sync_copy(dsts.at[gid, pl.ds(g * GC, GC)], dstb)
        descs = []
        for j in range(GC):
            descs.append(
                pltpu.async_copy(ones, acc.at[dstb.at[j]], sem0, add=True))
        for d in descs:
            d.wait()
    plsc.subcore_barrier()
    _dump_acc_stripe(acc, p, cid, row0)


def _init_body(xp, pdeg, h, dinv, pa, pb, xb, sem0, sem1, sem2):
    cid = lax.axis_index("c")
    wid = lax.axis_index("s")
    row0 = (cid * NS + wid) * RPW

    @pl.loop(0, RPW // 64)
    def _blk(i):
        r0 = row0 + i * 64
        d0 = pltpu.async_copy(pdeg.at[0, pl.ds(r0, 64)], pa, sem0)
        d1 = pltpu.async_copy(pdeg.at[1, pl.ds(r0, 64)], pb, sem1)
        d2 = pltpu.async_copy(xp.at[pl.ds(r0, 64)], xb, sem2)
        d0.wait()
        d1.wait()
        d2.wait()

        @pl.loop(0, 64)
        def _cv(r):
            dv = _rsqrt16(pa[r, pl.ds(0, L)] + pb[r, pl.ds(0, L)])
            for s in range(8):
                sl = pl.ds(s * L, L)
                pa[r, sl] = dv
                xb[r, sl] = xb[r, sl] * dv

        d3 = pltpu.async_copy(pa, dinv.at[pl.ds(r0, 64)], sem0)
        d4 = pltpu.async_copy(xb, h.at[pl.ds(r0, 64)], sem1)
        d3.wait()
        d4.wait()


def _edge_body(h_hbm, srcs, dsts, p,
               acc, srcb, dstb, msga, msgb, zbuf,
               gsem0, gsem1, ssem0, ssem1):
    cid = lax.axis_index("c")
    wid = lax.axis_index("s")
    gid = cid * NS + wid
    row0 = wid * RPT

    _fill(zbuf, 16, 0.0)
    _zero_acc_stripe(acc, zbuf, row0)
    plsc.subcore_barrier()

    @pl.loop(0, NG)
    def _edge(g):
        pltpu.sync_copy(srcs.at[gid, pl.ds(g * (GC * 128), GC * 128)], srcb)
        pltpu.sync_copy(dsts.at[gid, pl.ds(g * GC, GC)], dstb)
        bufs = (msga, msgb)
        gsems = (gsem0, gsem1)
        ssems = (ssem0, ssem1)
        gd = [None, None]
        sd = [None, None]
        for j in range(GC):
            b = j & 1
            if sd[b] is not None:
                sd[b].wait()
            idx = srcb.at[pl.ds(j * 128, 128)]
            gd[b] = pltpu.async_copy(h_hbm.at[idx], bufs[b], gsems[b])
            if j >= 1:
                pb = (j - 1) & 1
                gd[pb].wait()
                sd[pb] = pltpu.async_copy(
                    bufs[pb], acc.at[dstb.at[j - 1]], ssems[pb], add=True)
        lb = (GC - 1) & 1
        gd[lb].wait()
        sd[lb] = pltpu.async_copy(
            bufs[lb], acc.at[dstb.at[GC - 1]], ssems[lb], add=True)
        sd[0].wait()
        sd[1].wait()
    plsc.subcore_barrier()
    _dump_acc_stripe(acc, p, cid, row0)


def _comb_body(xp, dinv, p, y_hbm, h_hbm, pa, pb, dv, xb,
               sem0, sem1, sem2, sem3):
    cid = lax.axis_index("c")
    wid = lax.axis_index("s")
    row0 = (cid * NS + wid) * RPW

    @pl.loop(0, RPW // 64)
    def _blk(i):
        r0 = row0 + i * 64
        d0 = pltpu.async_copy(p.at[0, pl.ds(r0, 64)], pa, sem0)
        d1 = pltpu.async_copy(p.at[1, pl.ds(r0, 64)], pb, sem1)
        d2 = pltpu.async_copy(dinv.at[pl.ds(r0, 64)], dv, sem2)
        d3 = pltpu.async_copy(xp.at[pl.ds(r0, 64)], xb, sem3)
        d0.wait()
        d1.wait()
        d2.wait()
        d3.wait()

        @pl.loop(0, 64)
        def _cv(r):
            for s in range(8):
                sl = pl.ds(s * L, L)
                yv = (0.5 * ((pa[r, sl] + pb[r, sl]) * dv[r, sl])
                      + 0.5 * xb[r, sl])
                xb[r, sl] = yv
                pa[r, sl] = yv * dv[r, sl]

        d4 = pltpu.async_copy(xb, y_hbm.at[pl.ds(r0, 64)], sem0)
        d5 = pltpu.async_copy(pa, h_hbm.at[pl.ds(r0, 64)], sem1)
        d4.wait()
        d5.wait()


_deg_call = pl.kernel(
    _deg_body,
    out_type=jax.ShapeDtypeStruct((NC, NP, D), f32),
    mesh=_MESH,
    scratch_types=[
        pltpu.VMEM_SHARED((NP, D), f32),
        pltpu.VMEM((GC, 128), jnp.int32),
        pltpu.VMEM((128, D), f32),
        pltpu.VMEM((16, D), f32),
        pltpu.SemaphoreType.DMA,
    ],
)

_init_call = pl.kernel(
    _init_body,
    out_type=(
        jax.ShapeDtypeStruct((NP, D), f32),
        jax.ShapeDtypeStruct((NP, D), f32),
    ),
    mesh=_MESH,
    scratch_types=[
        pltpu.VMEM((64, D), f32),
        pltpu.VMEM((64, D), f32),
        pltpu.VMEM((64, D), f32),
        pltpu.SemaphoreType.DMA,
        pltpu.SemaphoreType.DMA,
        pltpu.SemaphoreType.DMA,
    ],
)

_edge_call = pl.kernel(
    _edge_body,
    out_type=jax.ShapeDtypeStruct((NC, NP, D), f32),
    mesh=_MESH,
    scratch_types=[
        pltpu.VMEM_SHARED((NP, D), f32),
        pltpu.VMEM((GC * 128,), jnp.int32),
        pltpu.VMEM((GC, 128), jnp.int32),
        pltpu.VMEM((128, D), f32),
        pltpu.VMEM((128, D), f32),
        pltpu.VMEM((16, D), f32),
        pltpu.SemaphoreType.DMA,
        pltpu.SemaphoreType.DMA,
        pltpu.SemaphoreType.DMA,
        pltpu.SemaphoreType.DMA,
    ],
)

_comb_call = pl.kernel(
    _comb_body,
    out_type=(
        jax.ShapeDtypeStruct((NP, D), f32),
        jax.ShapeDtypeStruct((NP, D), f32),
    ),
    mesh=_MESH,
    scratch_types=[
        pltpu.VMEM((64, D), f32),
        pltpu.VMEM((64, D), f32),
        pltpu.VMEM((64, D), f32),
        pltpu.VMEM((64, D), f32),
        pltpu.SemaphoreType.DMA,
        pltpu.SemaphoreType.DMA,
        pltpu.SemaphoreType.DMA,
        pltpu.SemaphoreType.DMA,
    ],
)


def kernel(x, edge_index):
    src = edge_index[0].reshape(NW, ET)
    dst = edge_index[1].reshape(NW, ET)
    pad = EP - ET
    src = jnp.pad(src, ((0, 0), (0, pad)))
    dst = jnp.pad(dst, ((0, 0), (0, pad)), constant_values=N)
    dst = dst.reshape(NW, NCH, 128)
    xp = jnp.pad(x, ((0, NP - N), (0, 0)))

    pdeg = _deg_call(dst)
    h, dinv = _init_call(xp, pdeg)
    y = None
    for _ in range(STEPS):
        p = _edge_call(h, src, dst)
        y, h = _comb_call(xp, dinv, p)
    return y[:N]


# speedup vs baseline: 3.6991x; 1.0180x over previous
import jax
import jax.numpy as jnp
from jax import lax
from jax.experimental import pallas as pl
from jax.experimental.pallas import tpu as pltpu
from jax.experimental.pallas import tpu_sc as plsc

N = 10000
D = 128
E = 320000
NC = 2
NS = 16
NW = NC * NS
NP = 10240
RPT = NP // NS
RPW = NP // NW
ET = E // NW
EP = 10240
NCH = EP // 128
GC = 16
NG = NCH // GC
STEPS = 5
L = 16

_MESH = plsc.VectorSubcoreMesh(core_axis_name="c", subcore_axis_name="s")
f32 = jnp.float32


def _fill(buf, rows, val):
    v = jnp.full((L,), val, f32)

    @pl.loop(0, rows)
    def _f(r):
        for s in range(8):
            buf[r, pl.ds(s * L, L)] = v


def _zero_acc_stripe(acc, zbuf, row0):
    @pl.loop(0, RPT // 16)
    def _z0(i):
        pltpu.sync_copy(zbuf, acc.at[pl.ds(row0 + i * 16, 16)])


def _dump_acc_stripe(acc, p, cid, row0):
    @pl.loop(0, RPT // 128)
    def _dmp(i):
        r0 = row0 + i * 128
        pltpu.sync_copy(acc.at[pl.ds(r0, 128)], p.at[cid, pl.ds(r0, 128)])


def _deg_body(dsts, p, acc, dstb, ones, zbuf, sem0, sem1):
    cid = lax.axis_index("c")
    wid = lax.axis_index("s")
    gid = cid * NS + wid
    row0 = wid * RPT

    _fill(zbuf, 16, 0.0)
    _fill(ones, 128, 1.0)
    _zero_acc_stripe(acc, zbuf, row0)
    plsc.subcore_barrier()

    @pl.loop(0, NG)
    def _deg(g):
        pltpu.sync_copy(dsts.at[gid, pl.ds(g * GC, GC)], dstb)
        sems = (sem0, sem1)
        sd = [None, None]
        for j in range(GC):
            b = j & 1
            if sd[b] is not None:
                sd[b].wait()
            sd[b] = pltpu.async_copy(ones, acc.at[dstb.at[j]], sems[b],
                                     add=True)
        sd[0].wait()
        sd[1].wait()
    plsc.subcore_barrier()
    _dump_acc_stripe(acc, p, cid, row0)


def _tc_init_body(x_ref, p0_ref, p1_ref, h_ref, dinv_ref):
    dv = lax.rsqrt(p0_ref[...] + p1_ref[...])
    dinv_ref[...] = dv
    h_ref[...] = x_ref[...] * dv


def _tc_comb_body(x_ref, dinv_ref, p0_ref, p1_ref, y_ref, h_ref):
    dv = dinv_ref[...]
    yv = 0.5 * ((p0_ref[...] + p1_ref[...]) * dv) + 0.5 * x_ref[...]
    y_ref[...] = yv
    h_ref[...] = yv * dv


def _edge_body(h_hbm, srcs, dsts, p,
               acc, srcb, dstb, msga, msgb, zbuf,
               gsem0, gsem1, ssem0, ssem1):
    cid = lax.axis_index("c")
    wid = lax.axis_index("s")
    gid = cid * NS + wid
    row0 = wid * RPT

    _fill(zbuf, 16, 0.0)
    _zero_acc_stripe(acc, zbuf, row0)
    plsc.subcore_barrier()

    @pl.loop(0, NG)
    def _edge(g):
        pltpu.sync_copy(srcs.at[gid, pl.ds(g * (GC * 128), GC * 128)], srcb)
        pltpu.sync_copy(dsts.at[gid, pl.ds(g * GC, GC)], dstb)
        bufs = (msga, msgb)
        gsems = (gsem0, gsem1)
        ssems = (ssem0, ssem1)
        gd = [None, None]
        sd = [None, None]
        for j in range(GC):
            b = j & 1
            if sd[b] is not None:
                sd[b].wait()
            idx = srcb.at[pl.ds(j * 128, 128)]
            gd[b] = pltpu.async_copy(h_hbm.at[idx], bufs[b], gsems[b])
            if j >= 1:
                pb = (j - 1) & 1
                gd[pb].wait()
                sd[pb] = pltpu.async_copy(
                    bufs[pb], acc.at[dstb.at[j - 1]], ssems[pb], add=True)
        lb = (GC - 1) & 1
        gd[lb].wait()
        sd[lb] = pltpu.async_copy(
            bufs[lb], acc.at[dstb.at[GC - 1]], ssems[lb], add=True)
        sd[0].wait()
        sd[1].wait()
    plsc.subcore_barrier()
    _dump_acc_stripe(acc, p, cid, row0)


_deg_call = pl.kernel(
    _deg_body,
    out_type=jax.ShapeDtypeStruct((NC, NP, D), f32),
    mesh=_MESH,
    scratch_types=[
        pltpu.VMEM_SHARED((NP, D), f32),
        pltpu.VMEM((GC, 128), jnp.int32),
        pltpu.VMEM((128, D), f32),
        pltpu.VMEM((16, D), f32),
        pltpu.SemaphoreType.DMA,
        pltpu.SemaphoreType.DMA,
    ],
)

TB = 512

_init_call = pl.pallas_call(
    _tc_init_body,
    grid=(NP // TB,),
    in_specs=[pl.BlockSpec((TB, D), lambda i: (i, 0))] * 3,
    out_specs=[pl.BlockSpec((TB, D), lambda i: (i, 0))] * 2,
    out_shape=(
        jax.ShapeDtypeStruct((NP, D), f32),
        jax.ShapeDtypeStruct((NP, D), f32),
    ),
)

_edge_call = pl.kernel(
    _edge_body,
    out_type=jax.ShapeDtypeStruct((NC, NP, D), f32),
    mesh=_MESH,
    scratch_types=[
        pltpu.VMEM_SHARED((NP, D), f32),
        pltpu.VMEM((GC * 128,), jnp.int32),
        pltpu.VMEM((GC, 128), jnp.int32),
        pltpu.VMEM((128, D), f32),
        pltpu.VMEM((128, D), f32),
        pltpu.VMEM((16, D), f32),
        pltpu.SemaphoreType.DMA,
        pltpu.SemaphoreType.DMA,
        pltpu.SemaphoreType.DMA,
        pltpu.SemaphoreType.DMA,
    ],
)

_comb_call = pl.pallas_call(
    _tc_comb_body,
    grid=(NP // TB,),
    in_specs=[pl.BlockSpec((TB, D), lambda i: (i, 0))] * 4,
    out_specs=[pl.BlockSpec((TB, D), lambda i: (i, 0))] * 2,
    out_shape=(
        jax.ShapeDtypeStruct((NP, D), f32),
        jax.ShapeDtypeStruct((NP, D), f32),
    ),
)


def kernel(x, edge_index):
    src = edge_index[0].reshape(NW, ET)
    dst = edge_index[1].reshape(NW, ET)
    pad = EP - ET
    src = jnp.pad(src, ((0, 0), (0, pad)))
    dst = jnp.pad(dst, ((0, 0), (0, pad)), constant_values=N)
    dst = dst.reshape(NW, NCH, 128)
    xp = jnp.pad(x, ((0, NP - N), (0, 0)))

    pdeg = _deg_call(dst)
    h, dinv = _init_call(xp, pdeg[0], pdeg[1])
    y = None
    for _ in range(STEPS):
        p = _edge_call(h, src, dst)
        y, h = _comb_call(xp, dinv, p[0], p[1])
    return y[:N]
